# Initial kernel scaffold; baseline (speedup 1.0000x reference)
import math

import jax
import jax.numpy as jnp
from jax import lax
from jax.experimental import pallas as pl
from jax.experimental.pallas import tpu as pltpu
from jax.experimental.pallas import tpu_sc as plsc

NN, EE, NB, HH, AA = 10000, 320000, 16, 128, 100
NC, NS = 2, 16
NW = NC * NS
EPW = EE // NW
CH = 80
NCHUNK = EPW // CH
RPT = NN // NS
RB = 1000
NRB = NN // RB

F32 = jnp.float32


def _dot(a, b):
    return lax.dot_general(a, b, (((1,), (0,)), ((), ())),
                           precision=lax.Precision.HIGHEST,
                           preferred_element_type=F32)


def _silu(x):
    return x * jax.nn.sigmoid(x)


_MESH = plsc.VectorSubcoreMesh(core_axis_name="c", subcore_axis_name="s",
                               num_cores=NC, num_subcores=NS)


def _prep_body(pos16, colh, rowh, disth, cnth,
               icol, irow, pA, pB, dbuf, cntv, sem1, sem2):
    c = lax.axis_index("c")
    s = lax.axis_index("s")
    w = c * NS + s
    zeros16 = jnp.zeros((16,), F32)
    ones16 = jnp.ones((16,), F32)
    lane = lax.iota(jnp.int32, 16)
    mask3 = jnp.where(lane < 3, 1.0, 0.0).astype(F32)

    def zc(i, carry):
        cntv[pl.ds(i * 16, 16)] = zeros16
        return carry
    lax.fori_loop(0, NN // 16, zc, 0)

    ebase = w * EPW

    def chunk(g, carry):
        b = ebase + g * CH
        pltpu.sync_copy(colh.at[pl.ds(b, CH)], icol)
        pltpu.sync_copy(rowh.at[pl.ds(b, CH)], irow)
        cp1 = pltpu.async_copy(pos16.at[irow], pA, sem1)
        cp2 = pltpu.async_copy(pos16.at[icol], pB, sem2)
        cp1.wait()
        cp2.wait()

        def edge(e, cc):
            u = (pA[e, :] - pB[e, :]) * mask3
            dbuf[e] = jnp.sum(u * u)
            return cc
        lax.fori_loop(0, CH, edge, 0)

        for k in range(CH // 16):
            iv = icol[pl.ds(k * 16, 16)]
            plsc.addupdate_scatter(cntv, [iv], ones16)

        for k in range(CH // 16):
            sl = pl.ds(k * 16, 16)
            v = dbuf[sl]
            y = plsc.bitcast(v, jnp.int32)
            y = 0x5F3759DF - (y >> 1)
            r = plsc.bitcast(y, F32)
            for _ in range(3):
                r = r * (1.5 - 0.5 * v * r * r)
            dbuf[sl] = v * r
        pltpu.sync_copy(dbuf, disth.at[pl.ds(b, CH)])
        return carry
    lax.fori_loop(0, NCHUNK, chunk, 0)
    pltpu.sync_copy(cntv, cnth.at[w])


_prep_call = pl.kernel(
    _prep_body,
    out_type=(jax.ShapeDtypeStruct((EE,), F32),
              jax.ShapeDtypeStruct((NW, NN), F32)),
    mesh=_MESH,
    scratch_types=[
        pltpu.VMEM((CH,), jnp.int32),
        pltpu.VMEM((CH,), jnp.int32),
        pltpu.VMEM((CH, 16), F32),
        pltpu.VMEM((CH, 16), F32),
        pltpu.VMEM((CH,), F32),
        pltpu.VMEM((NN,), F32),
        pltpu.SemaphoreType.DMA,
        pltpu.SemaphoreType.DMA,
    ],
)


def _edge_body(pih, pjh, colh, rowh, disth, w1ch, outh,
               icol, irow, dvec, piB, pjB, w1cv, zbuf, S_sh, sem1, sem2):
    c = lax.axis_index("c")
    s = lax.axis_index("s")
    w = c * NS + s
    zeros16 = jnp.zeros((16,), F32)
    pltpu.sync_copy(w1ch, w1cv)

    def zb(i, carry):
        for k in range(HH // 16):
            zbuf[i, pl.ds(k * 16, 16)] = zeros16
        return carry
    lax.fori_loop(0, 125, zb, 0)
    for t in range(RPT // 125):
        pltpu.sync_copy(zbuf, S_sh.at[pl.ds(s * RPT + t * 125, 125)])
    plsc.subcore_barrier()

    ebase = w * EPW

    def chunk(g, carry):
        b = ebase + g * CH
        pltpu.sync_copy(colh.at[pl.ds(b, CH)], icol)
        pltpu.sync_copy(rowh.at[pl.ds(b, CH)], irow)
        pltpu.sync_copy(disth.at[pl.ds(b, CH)], dvec)
        cp1 = pltpu.async_copy(pih.at[icol], piB, sem1)
        cp2 = pltpu.async_copy(pjh.at[irow], pjB, sem2)
        cp1.wait()
        cp2.wait()

        def edge(e, cc):
            sv = jnp.full((16,), dvec[e], F32)
            for k in range(HH // 16):
                sl = pl.ds(k * 16, 16)
                a = piB[e, sl] + pjB[e, sl] + sv * w1cv[sl]
                piB[e, sl] = a / (1.0 + jnp.exp(-a))
            return cc
        lax.fori_loop(0, CH, edge, 0)
        pltpu.sync_copy(piB, S_sh.at[icol], add=True)
        return carry
    lax.fori_loop(0, NCHUNK, chunk, 0)
    plsc.subcore_barrier()
    pltpu.sync_copy(S_sh.at[pl.ds(s * RPT, RPT)],
                    outh.at[c, pl.ds(s * RPT, RPT)])


_edge_call = pl.kernel(
    _edge_body,
    out_type=jax.ShapeDtypeStruct((NC, NN, HH), F32),
    mesh=_MESH,
    scratch_types=[
        pltpu.VMEM((CH,), jnp.int32),
        pltpu.VMEM((CH,), jnp.int32),
        pltpu.VMEM((CH,), F32),
        pltpu.VMEM((CH, HH), F32),
        pltpu.VMEM((CH, HH), F32),
        pltpu.VMEM((HH,), F32),
        pltpu.VMEM((125, HH), F32),
        pltpu.VMEM_SHARED((NN, HH), F32),
        pltpu.SemaphoreType.DMA,
        pltpu.SemaphoreType.DMA,
    ],
)


def _tp_body(ts, cond, tw1, tb1, tw2, tb2, cw1, cb1, cw2, cb2,
             w10, b10, w20, b20, w11, b11, w21, b21, tp0, tp1):
    t = ts[...].astype(F32)
    half = HH // 2
    k = lax.broadcasted_iota(F32, (1, half), 1)
    freq = jnp.exp(k * (-(math.log(10000.0) / (half - 1))))
    te = t * freq
    te = jnp.concatenate([jnp.sin(te), jnp.cos(te)], axis=-1)
    te = _dot(_silu(_dot(te, tw1[...]) + tb1[...]), tw2[...]) + tb2[...]
    ce = _dot(_silu(_dot(cond[...], cw1[...]) + cb1[...]), cw2[...]) + cb2[...]
    te = te + ce
    tp0[...] = _dot(_silu(_dot(te, w10[...]) + b10[...]), w20[...]) + b20[...]
    tp1[...] = _dot(_silu(_dot(te, w11[...]) + b11[...]), w21[...]) + b21[...]


def _embed_body(xr, br, emb, tp0, w1aT, b1e, w1bT, ho, pio, pjo):
    oh = (xr[...] == lax.broadcasted_iota(jnp.int32, (RB, AA), 1)).astype(F32)
    h = _dot(oh, emb[...])
    ohb = (br[...] == lax.broadcasted_iota(jnp.int32, (RB, NB), 1)).astype(F32)
    h = h + _dot(ohb, tp0[...])
    ho[...] = h
    pio[...] = _dot(h, w1aT[...]) + b1e[...]
    pjo[...] = _dot(h, w1bT[...])


def _make_node_mid(has_tp):
    def body(hr, s0, s1, r1, r2, w2T, b2e, nw1T, nb1, nw2T, nb2,
             lng, lnb, *rest):
        if has_tp:
            tpn, br, w1aT, b1e, w1bT, ho, pio, pjo = rest
        else:
            w1aT, b1e, w1bT, ho, pio, pjo = rest
        h = hr[...]
        S = s0[...] + s1[...]
        agg = _dot(S, w2T[...]) * r1[...] + r2[...] * b2e[...]
        t1 = _silu(_dot(h, nw1T[...]) + nb1[...])
        hm = _dot(t1, nw2T[...]) + nb2[...]
        u = h + hm + agg
        m = u.mean(-1, keepdims=True)
        v = ((u - m) ** 2).mean(-1, keepdims=True)
        hO = (u - m) / jnp.sqrt(v + 1e-5) * lng[...] + lnb[...]
        if has_tp:
            ohb = (br[...] == lax.broadcasted_iota(jnp.int32, (RB, NB), 1)
                   ).astype(F32)
            hO = hO + _dot(ohb, tpn[...])
        ho[...] = hO
        pio[...] = _dot(hO, w1aT[...]) + b1e[...]
        pjo[...] = _dot(hO, w1bT[...])
    return body


def _node_last_body(hr, s0, s1, r1, r2, w2T, b2e, nw1T, nb1, nw2T, nb2,
                    lng, lnb, aw1T, ab1, aw2T, ab2, cw1T, cb1, cw2T, cb2,
                    atom_o, coord_o):
    h = hr[...]
    S = s0[...] + s1[...]
    agg = _dot(S, w2T[...]) * r1[...] + r2[...] * b2e[...]
    t1 = _silu(_dot(h, nw1T[...]) + nb1[...])
    hm = _dot(t1, nw2T[...]) + nb2[...]
    u = h + hm + agg
    m = u.mean(-1, keepdims=True)
    v = ((u - m) ** 2).mean(-1, keepdims=True)
    hO = (u - m) / jnp.sqrt(v + 1e-5) * lng[...] + lnb[...]
    atom_o[...] = _dot(_silu(_dot(hO, aw1T[...]) + ab1[...]), aw2T[...]) + ab2[...]
    coord_o[...] = _dot(_silu(_dot(hO, cw1T[...]) + cb1[...]), cw2T[...]) + cb2[...]


_LAYERS = [(0, 0), (0, 1), (1, 0), (1, 1)]


def _rowspec():
    return pl.BlockSpec((RB, HH), lambda i: (i, 0))


def _colspec():
    return pl.BlockSpec((RB, 1), lambda i: (i, 0))


def _fullspec(a):
    nd = a.ndim
    return pl.BlockSpec(a.shape, lambda i: (0,) * nd)


def kernel(x, pos, edge_index, batch, timesteps, conditions, params):
    P = params
    col = edge_index[1].astype(jnp.int32)
    row = edge_index[0].astype(jnp.int32)
    x2 = x.astype(jnp.int32).reshape(NN, 1)
    bcol = batch.astype(jnp.int32).reshape(NN, 1)
    ts2 = timesteps.astype(jnp.int32).reshape(NB, 1)
    pos16 = jnp.concatenate([pos.astype(F32), jnp.zeros((NN, 13), F32)], axis=1)

    dist, cnt32 = _prep_call(pos16, col, row)
    cnt = cnt32.sum(0)
    deg = jnp.maximum(cnt, 1.0)
    r1 = (1.0 / deg)[:, None]
    r2 = (cnt / deg)[:, None]

    tp_args = (
        ts2, conditions.astype(F32),
        P['time_w1'].T, P['time_b1'][None], P['time_w2'].T, P['time_b2'][None],
        P['cond_w1'].T, P['cond_b1'][None], P['cond_w2'].T, P['cond_b2'][None],
        P['b0_tp_w1'].T, P['b0_tp_b1'][None], P['b0_tp_w2'].T, P['b0_tp_b2'][None],
        P['b1_tp_w1'].T, P['b1_tp_b1'][None], P['b1_tp_w2'].T, P['b1_tp_b2'][None],
    )
    tp0, tp1 = pl.pallas_call(
        _tp_body,
        out_shape=(jax.ShapeDtypeStruct((NB, HH), F32),
                   jax.ShapeDtypeStruct((NB, HH), F32)),
    )(*tp_args)

    def edge_w(idx):
        bi, li = _LAYERS[idx]
        p = 'b%d_l%d_' % (bi, li)
        W1 = P[p + 'edge_w1']
        return (W1[:, :HH].T, P[p + 'edge_b1'][None], W1[:, HH:2 * HH].T,
                W1[:, 2 * HH])

    w1aT0, b1e0, w1bT0, _ = edge_w(0)
    emb = P['atom_emb']
    h, Pi, Pj = pl.pallas_call(
        _embed_body,
        grid=(NRB,),
        in_specs=[_colspec(), _colspec(), _fullspec(emb), _fullspec(tp0),
                  _fullspec(w1aT0), _fullspec(b1e0), _fullspec(w1bT0)],
        out_specs=[_rowspec()] * 3,
        out_shape=(jax.ShapeDtypeStruct((NN, HH), F32),) * 3,
    )(x2, bcol, emb, tp0, w1aT0, b1e0, w1bT0)

    atom = coord = None
    for idx, (bi, li) in enumerate(_LAYERS):
        p = 'b%d_l%d_' % (bi, li)
        w1c = edge_w(idx)[3]
        S2 = _edge_call(Pi, Pj, col, row, dist, w1c)
        common = [h, S2[0], S2[1], r1, r2,
                  P[p + 'edge_w2'].T, P[p + 'edge_b2'][None],
                  P[p + 'node_w1'].T, P[p + 'node_b1'][None],
                  P[p + 'node_w2'].T, P[p + 'node_b2'][None],
                  P[p + 'ln_g'][None], P[p + 'ln_b'][None]]
        cspecs = [_rowspec(), _rowspec(), _rowspec(), _colspec(), _colspec()] + \
                 [_fullspec(a) for a in common[5:]]
        if idx < 3:
            nbi, nli = _LAYERS[idx + 1]
            has_tp = (nbi == 1 and nli == 0)
            w1aTn, b1en, w1bTn, _ = edge_w(idx + 1)
            args = list(common)
            specs = list(cspecs)
            if has_tp:
                args += [tp1, bcol]
                specs += [_fullspec(tp1), _colspec()]
            args += [w1aTn, b1en, w1bTn]
            specs += [_fullspec(w1aTn), _fullspec(b1en), _fullspec(w1bTn)]
            h, Pi, Pj = pl.pallas_call(
                _make_node_mid(has_tp),
                grid=(NRB,),
                in_specs=specs,
                out_specs=[_rowspec()] * 3,
                out_shape=(jax.ShapeDtypeStruct((NN, HH), F32),) * 3,
            )(*args)
        else:
            head = [P['out_atom_w1'].T, P['out_atom_b1'][None],
                    P['out_atom_w2'].T, P['out_atom_b2'][None],
                    P['out_coord_w1'].T, P['out_coord_b1'][None],
                    P['out_coord_w2'].T, P['out_coord_b2'][None]]
            args = common + head
            specs = cspecs + [_fullspec(a) for a in head]
            atom, coord = pl.pallas_call(
                _node_last_body,
                grid=(NRB,),
                in_specs=specs,
                out_specs=[pl.BlockSpec((RB, AA), lambda i: (i, 0)),
                           pl.BlockSpec((RB, 3), lambda i: (i, 0))],
                out_shape=(jax.ShapeDtypeStruct((NN, AA), F32),
                           jax.ShapeDtypeStruct((NN, 3), F32)),
            )(*args)
    return (atom, coord)


# speedup vs baseline: 1.2921x; 1.2921x over previous
import math

import jax
import jax.numpy as jnp
from jax import lax
from jax.experimental import pallas as pl
from jax.experimental.pallas import tpu as pltpu
from jax.experimental.pallas import tpu_sc as plsc

NN, EE, NB, HH, AA = 10000, 320000, 16, 128, 100
NC, NS = 2, 16
NW = NC * NS
EPW = EE // NW
CH = 80
NCHUNK = EPW // CH
NPAD = 10240
RPT = NPAD // NS
RB = 1000
NRB = NN // RB

F32 = jnp.float32


def _dot(a, b):
    return lax.dot_general(a, b, (((1,), (0,)), ((), ())),
                           precision=lax.Precision.HIGHEST,
                           preferred_element_type=F32)


def _silu(x):
    return x * jax.nn.sigmoid(x)


_SC_CACHE = {}


def _sc_mesh():
    if "mesh" not in _SC_CACHE:
        _SC_CACHE["mesh"] = plsc.VectorSubcoreMesh(
            core_axis_name="c", subcore_axis_name="s",
            num_cores=NC, num_subcores=NS)
    return _SC_CACHE["mesh"]


def _prep_body(pos4h, colh, rowh, disth, cnth,
               icol, irow, posv, dbuf, cntv):
    c = lax.axis_index("c")
    s = lax.axis_index("s")
    w = c * NS + s
    zeros16 = jnp.zeros((16,), F32)
    ones16 = jnp.ones((16,), F32)

    pltpu.sync_copy(pos4h, posv)

    def zc(i, carry):
        for k in range(HH // 16):
            cntv[i, pl.ds(k * 16, 16)] = zeros16
        return carry
    lax.fori_loop(0, NPAD // 128, zc, 0)

    ebase = w * EPW

    def gat(iv, comp):
        t = (iv << 2) + comp
        return plsc.load_gather(posv, [t >> 7, t & 127])

    def chunk(g, carry):
        b = ebase + g * CH
        pltpu.sync_copy(colh.at[pl.ds(b, CH)], icol)
        pltpu.sync_copy(rowh.at[pl.ds(b, CH)], irow)
        for k in range(CH // 16):
            sl = pl.ds(k * 16, 16)
            ir = irow[sl]
            ic = icol[sl]
            dx = gat(ir, 0) - gat(ic, 0)
            dy = gat(ir, 1) - gat(ic, 1)
            dz = gat(ir, 2) - gat(ic, 2)
            v = dx * dx + dy * dy + dz * dz
            y = plsc.bitcast(v, jnp.int32)
            y = 0x5F3759DF - (y >> 1)
            r = plsc.bitcast(y, F32)
            for _ in range(3):
                r = r * (1.5 - 0.5 * v * r * r)
            dbuf[sl] = v * r
            plsc.addupdate_scatter(cntv, [ic >> 7, ic & 127], ones16)
        pltpu.sync_copy(dbuf, disth.at[pl.ds(b, CH)])
        return carry
    lax.fori_loop(0, NCHUNK, chunk, 0)
    pltpu.sync_copy(cntv, cnth.at[w])


def _prep_call(*args):
    if "prep" not in _SC_CACHE:
        _SC_CACHE["prep"] = pl.kernel(
            _prep_body,
            out_type=(jax.ShapeDtypeStruct((EE,), F32),
                      jax.ShapeDtypeStruct((NW, NPAD // 128, 128), F32)),
            mesh=_sc_mesh(),
            compiler_params=pltpu.CompilerParams(needs_layout_passes=False),
            scratch_types=[
                pltpu.VMEM((CH,), jnp.int32),
                pltpu.VMEM((CH,), jnp.int32),
                pltpu.VMEM((NPAD * 4 // 128, 128), F32),
                pltpu.VMEM((CH,), F32),
                pltpu.VMEM((NPAD // 128, 128), F32),
            ],
        )
    return _SC_CACHE["prep"](*args)


def _edge_body(pih, pjh, colh, rowh, disth, w1ch, outh,
               icol, irow, dvec, piB, pjB, w1cv, zbuf, S_sh, sem1, sem2):
    c = lax.axis_index("c")
    s = lax.axis_index("s")
    w = c * NS + s
    zeros16 = jnp.zeros((16,), F32)
    pltpu.sync_copy(w1ch, w1cv)

    def zb(i, carry):
        for k in range(HH // 16):
            zbuf[i, pl.ds(k * 16, 16)] = zeros16
        return carry
    lax.fori_loop(0, 128, zb, 0)
    for t in range(RPT // 128):
        pltpu.sync_copy(zbuf, S_sh.at[pl.ds(s * RPT + t * 128, 128)])
    plsc.subcore_barrier()

    ebase = w * EPW

    def chunk(g, carry):
        b = ebase + g * CH
        pltpu.sync_copy(colh.at[pl.ds(b, CH)], icol)
        pltpu.sync_copy(rowh.at[pl.ds(b, CH)], irow)
        pltpu.sync_copy(disth.at[pl.ds(b, CH)], dvec.at[pl.ds(0, CH)])
        cp1 = pltpu.async_copy(pih.at[icol], piB, sem1)
        cp2 = pltpu.async_copy(pjh.at[irow], pjB, sem2)
        cp1.wait()
        cp2.wait()

        def edge(e, cc):
            sv = jnp.full((16,), dvec[pl.ds(e, 16)][0], F32)
            for k in range(HH // 16):
                sl = pl.ds(k * 16, 16)
                a = piB[e, sl] + pjB[e, sl] + sv * w1cv[sl]
                piB[e, sl] = a / (1.0 + jnp.exp(-a))
            return cc
        lax.fori_loop(0, CH, edge, 0)
        pltpu.sync_copy(piB, S_sh.at[icol], add=True)
        return carry
    lax.fori_loop(0, NCHUNK, chunk, 0)
    plsc.subcore_barrier()
    pltpu.sync_copy(S_sh.at[pl.ds(s * RPT, RPT)],
                    outh.at[c, pl.ds(s * RPT, RPT)])


def _edge_call(*args):
    if "edge" not in _SC_CACHE:
        _SC_CACHE["edge"] = pl.kernel(
            _edge_body,
            out_type=jax.ShapeDtypeStruct((NC, NPAD, HH), F32),
            mesh=_sc_mesh(),
            compiler_params=pltpu.CompilerParams(needs_layout_passes=False),
            scratch_types=[
                pltpu.VMEM((CH,), jnp.int32),
                pltpu.VMEM((CH,), jnp.int32),
                pltpu.VMEM((CH + 16,), F32),
                pltpu.VMEM((CH, HH), F32),
                pltpu.VMEM((CH, HH), F32),
                pltpu.VMEM((HH,), F32),
                pltpu.VMEM((128, HH), F32),
                pltpu.VMEM_SHARED((NPAD, HH), F32),
                pltpu.SemaphoreType.DMA,
                pltpu.SemaphoreType.DMA,
            ],
        )
    return _SC_CACHE["edge"](*args)


def _tp_body(ts, cond, tw1, tb1, tw2, tb2, cw1, cb1, cw2, cb2,
             w10, b10, w20, b20, w11, b11, w21, b21, tp0, tp1):
    t = ts[...].astype(F32)
    half = HH // 2
    k = lax.broadcasted_iota(jnp.int32, (1, half), 1).astype(F32)
    freq = jnp.exp(k * (-(math.log(10000.0) / (half - 1))))
    te = t * freq
    te = jnp.concatenate([jnp.sin(te), jnp.cos(te)], axis=-1)
    te = _dot(_silu(_dot(te, tw1[...]) + tb1[...]), tw2[...]) + tb2[...]
    ce = _dot(_silu(_dot(cond[...], cw1[...]) + cb1[...]), cw2[...]) + cb2[...]
    te = te + ce
    tp0[...] = _dot(_silu(_dot(te, w10[...]) + b10[...]), w20[...]) + b20[...]
    tp1[...] = _dot(_silu(_dot(te, w11[...]) + b11[...]), w21[...]) + b21[...]


def _embed_body(xr, br, emb, tp0, w1aT, b1e, w1bT, ho, pio, pjo):
    oh = (xr[...] == lax.broadcasted_iota(jnp.int32, (RB, AA), 1)).astype(F32)
    h = _dot(oh, emb[...])
    ohb = (br[...] == lax.broadcasted_iota(jnp.int32, (RB, NB), 1)).astype(F32)
    h = h + _dot(ohb, tp0[...])
    ho[...] = h
    pio[...] = _dot(h, w1aT[...]) + b1e[...]
    pjo[...] = _dot(h, w1bT[...])


def _make_node_mid(has_tp):
    def body(hr, s0, s1, r1, r2, w2T, b2e, nw1T, nb1, nw2T, nb2,
             lng, lnb, *rest):
        if has_tp:
            tpn, br, w1aT, b1e, w1bT, ho, pio, pjo = rest
        else:
            w1aT, b1e, w1bT, ho, pio, pjo = rest
        h = hr[...]
        S = s0[...] + s1[...]
        agg = _dot(S, w2T[...]) * r1[...] + r2[...] * b2e[...]
        t1 = _silu(_dot(h, nw1T[...]) + nb1[...])
        hm = _dot(t1, nw2T[...]) + nb2[...]
        u = h + hm + agg
        m = u.mean(-1, keepdims=True)
        v = ((u - m) ** 2).mean(-1, keepdims=True)
        hO = (u - m) / jnp.sqrt(v + 1e-5) * lng[...] + lnb[...]
        if has_tp:
            ohb = (br[...] == lax.broadcasted_iota(jnp.int32, (RB, NB), 1)
                   ).astype(F32)
            hO = hO + _dot(ohb, tpn[...])
        ho[...] = hO
        pio[...] = _dot(hO, w1aT[...]) + b1e[...]
        pjo[...] = _dot(hO, w1bT[...])
    return body


def _node_last_body(hr, s0, s1, r1, r2, w2T, b2e, nw1T, nb1, nw2T, nb2,
                    lng, lnb, aw1T, ab1, aw2T, ab2, cw1T, cb1, cw2T, cb2,
                    atom_o, coord_o):
    h = hr[...]
    S = s0[...] + s1[...]
    agg = _dot(S, w2T[...]) * r1[...] + r2[...] * b2e[...]
    t1 = _silu(_dot(h, nw1T[...]) + nb1[...])
    hm = _dot(t1, nw2T[...]) + nb2[...]
    u = h + hm + agg
    m = u.mean(-1, keepdims=True)
    v = ((u - m) ** 2).mean(-1, keepdims=True)
    hO = (u - m) / jnp.sqrt(v + 1e-5) * lng[...] + lnb[...]
    atom_o[...] = _dot(_silu(_dot(hO, aw1T[...]) + ab1[...]), aw2T[...]) + ab2[...]
    coord_o[...] = _dot(_silu(_dot(hO, cw1T[...]) + cb1[...]), cw2T[...]) + cb2[...]


_LAYERS = [(0, 0), (0, 1), (1, 0), (1, 1)]


def _rowspec():
    return pl.BlockSpec((RB, HH), lambda i: (i, 0))


def _colspec():
    return pl.BlockSpec((RB, 1), lambda i: (i, 0))


def _fullspec(a):
    nd = a.ndim
    return pl.BlockSpec(a.shape, lambda i: (0,) * nd)


def kernel(x, pos, edge_index, batch, timesteps, conditions, params):
    P = params
    col = edge_index[1].astype(jnp.int32)
    row = edge_index[0].astype(jnp.int32)
    x2 = x.astype(jnp.int32).reshape(NN, 1)
    bcol = batch.astype(jnp.int32).reshape(NN, 1)
    ts2 = timesteps.astype(jnp.int32).reshape(NB, 1)
    pos4 = jnp.concatenate([pos.astype(F32), jnp.zeros((NN, 1), F32)], axis=1)
    pos4 = jnp.concatenate([pos4, jnp.zeros((NPAD - NN, 4), F32)], axis=0)
    pos4 = pos4.reshape(NPAD * 4 // 128, 128)

    dist, cnt32 = _prep_call(pos4, col, row)
    cnt = cnt32.reshape(NW, NPAD)[:, :NN].sum(0)
    deg = jnp.maximum(cnt, 1.0)
    r1 = (1.0 / deg)[:, None]
    r2 = (cnt / deg)[:, None]

    tp_args = (
        ts2, conditions.astype(F32),
        P['time_w1'].T, P['time_b1'][None], P['time_w2'].T, P['time_b2'][None],
        P['cond_w1'].T, P['cond_b1'][None], P['cond_w2'].T, P['cond_b2'][None],
        P['b0_tp_w1'].T, P['b0_tp_b1'][None], P['b0_tp_w2'].T, P['b0_tp_b2'][None],
        P['b1_tp_w1'].T, P['b1_tp_b1'][None], P['b1_tp_w2'].T, P['b1_tp_b2'][None],
    )
    tp0, tp1 = pl.pallas_call(
        _tp_body,
        out_shape=(jax.ShapeDtypeStruct((NB, HH), F32),
                   jax.ShapeDtypeStruct((NB, HH), F32)),
    )(*tp_args)

    def edge_w(idx):
        bi, li = _LAYERS[idx]
        p = 'b%d_l%d_' % (bi, li)
        W1 = P[p + 'edge_w1']
        return (W1[:, :HH].T, P[p + 'edge_b1'][None], W1[:, HH:2 * HH].T,
                W1[:, 2 * HH])

    w1aT0, b1e0, w1bT0, _ = edge_w(0)
    emb = P['atom_emb']
    h, Pi, Pj = pl.pallas_call(
        _embed_body,
        grid=(NRB,),
        in_specs=[_colspec(), _colspec(), _fullspec(emb), _fullspec(tp0),
                  _fullspec(w1aT0), _fullspec(b1e0), _fullspec(w1bT0)],
        out_specs=[_rowspec()] * 3,
        out_shape=(jax.ShapeDtypeStruct((NN, HH), F32),) * 3,
    )(x2, bcol, emb, tp0, w1aT0, b1e0, w1bT0)

    atom = coord = None
    for idx, (bi, li) in enumerate(_LAYERS):
        p = 'b%d_l%d_' % (bi, li)
        w1c = edge_w(idx)[3]
        S2 = _edge_call(Pi, Pj, col, row, dist, w1c)
        common = [h, S2[0], S2[1], r1, r2,
                  P[p + 'edge_w2'].T, P[p + 'edge_b2'][None],
                  P[p + 'node_w1'].T, P[p + 'node_b1'][None],
                  P[p + 'node_w2'].T, P[p + 'node_b2'][None],
                  P[p + 'ln_g'][None], P[p + 'ln_b'][None]]
        cspecs = [_rowspec(), _rowspec(), _rowspec(), _colspec(), _colspec()] + \
                 [_fullspec(a) for a in common[5:]]
        if idx < 3:
            nbi, nli = _LAYERS[idx + 1]
            has_tp = (nbi == 1 and nli == 0)
            w1aTn, b1en, w1bTn, _ = edge_w(idx + 1)
            args = list(common)
            specs = list(cspecs)
            if has_tp:
                args += [tp1, bcol]
                specs += [_fullspec(tp1), _colspec()]
            args += [w1aTn, b1en, w1bTn]
            specs += [_fullspec(w1aTn), _fullspec(b1en), _fullspec(w1bTn)]
            h, Pi, Pj = pl.pallas_call(
                _make_node_mid(has_tp),
                grid=(NRB,),
                in_specs=specs,
                out_specs=[_rowspec()] * 3,
                out_shape=(jax.ShapeDtypeStruct((NN, HH), F32),) * 3,
            )(*args)
        else:
            head = [P['out_atom_w1'].T, P['out_atom_b1'][None],
                    P['out_atom_w2'].T, P['out_atom_b2'][None],
                    P['out_coord_w1'].T, P['out_coord_b1'][None],
                    P['out_coord_w2'].T, P['out_coord_b2'][None]]
            args = common + head
            specs = cspecs + [_fullspec(a) for a in head]
            atom, coord = pl.pallas_call(
                _node_last_body,
                grid=(NRB,),
                in_specs=specs,
                out_specs=[pl.BlockSpec((RB, AA), lambda i: (i, 0)),
                           pl.BlockSpec((RB, 3), lambda i: (i, 0))],
                out_shape=(jax.ShapeDtypeStruct((NN, AA), F32),
                           jax.ShapeDtypeStruct((NN, 3), F32)),
            )(*args)
    return (atom, coord)
